# baseline (speedup 1.0000x reference)
import functools

import jax
import jax.numpy as jnp
from jax import lax
from jax.experimental import pallas as pl
from jax.experimental.pallas import tpu as pltpu
from jax.experimental.pallas import tpu_sc as plsc

B, K, C, L = 16, 8, 128, 4096
TOPK = 4
LB = 4096

_NEG = jnp.float32(-jnp.inf)


def _sc_routing_body(st_hbm, sel_hbm, w_hbm, st_v, sel_v, w_v):
    cid = lax.axis_index("c")
    sid = lax.axis_index("s")

    @pl.when(jnp.logical_and(cid == 0, sid == 0))
    def _():
        pltpu.sync_copy(st_hbm, st_v)
        cur = [st_v[k, :] for k in range(K)]
        vals = []
        idxs = []
        for _ in range(TOPK):
            m = cur[0]
            for k in range(1, K):
                m = jnp.maximum(m, cur[k])
            idx = jnp.full((B,), K, jnp.int32)
            for k in range(K - 1, -1, -1):
                idx = jnp.where(cur[k] == m,
                                jnp.full((B,), k, jnp.int32), idx)
            vals.append(m)
            idxs.append(idx)
            for k in range(K):
                cur[k] = jnp.where(idx == jnp.full((B,), k, jnp.int32),
                                   jnp.full((B,), _NEG), cur[k])
        tot = vals[0] + vals[1] + vals[2] + vals[3] + jnp.full((B,), 1e-8,
                                                              jnp.float32)
        for t in range(TOPK):
            sel_v[t, :] = idxs[t]
            w_v[t, :] = vals[t] / tot
        pltpu.sync_copy(sel_v, sel_hbm)
        pltpu.sync_copy(w_v, w_hbm)


_sc_routing = functools.partial(
    pl.kernel,
    mesh=plsc.VectorSubcoreMesh(core_axis_name="c", subcore_axis_name="s"),
    out_type=(
        jax.ShapeDtypeStruct((TOPK, B), jnp.int32),
        jax.ShapeDtypeStruct((TOPK, B), jnp.float32),
    ),
    scratch_types=[
        pltpu.VMEM((K, B), jnp.float32),
        pltpu.VMEM((TOPK, B), jnp.int32),
        pltpu.VMEM((TOPK, B), jnp.float32),
    ],
)(_sc_routing_body)


def _moe_body(sel_ref, w_ref, *refs):
    x_refs = refs[:TOPK]
    ew_ref = refs[TOPK]
    o_ref = refs[TOPK + 1]
    b = pl.program_id(0)
    acc = None
    for t in range(TOPK):
        w = w_ref[t, b]
        e = sel_ref[t, b]
        ew = (ew_ref[e] * w).astype(jnp.bfloat16)
        x = x_refs[t][0, 0].astype(jnp.bfloat16)
        d = jax.lax.dot_general(
            ew, x,
            dimension_numbers=(((0,), (0,)), ((), ())),
            preferred_element_type=jnp.float32,
        )
        acc = d if acc is None else acc + d
    o_ref[0] = acc


@jax.jit
def kernel(xs, scores, expert_weights):
    sel, w = _sc_routing(scores.T)

    n_l = L // LB
    grid = (B, n_l)

    def x_map(t):
        return lambda b, l, sel_ref, w_ref: (b, sel_ref[t, b], 0, l)

    in_specs = (
        [pl.BlockSpec((1, 1, C, LB), x_map(t)) for t in range(TOPK)]
        + [pl.BlockSpec((K, C, C), lambda b, l, sel_ref, w_ref: (0, 0, 0))]
    )
    out = pl.pallas_call(
        _moe_body,
        grid_spec=pltpu.PrefetchScalarGridSpec(
            num_scalar_prefetch=2,
            grid=grid,
            in_specs=in_specs,
            out_specs=pl.BlockSpec(
                (1, C, LB),
                lambda b, l, sel_ref, w_ref: (b, 0, l),
            ),
        ),
        out_shape=jax.ShapeDtypeStruct((B, C, L), jnp.float32),
    )(sel, w, *([xs] * TOPK), expert_weights)
    return out
